# baseline (speedup 1.0000x reference)
import functools

import jax
import jax.numpy as jnp
from jax import lax
from jax.experimental import pallas as pl
from jax.experimental.pallas import tpu as pltpu
from jax.experimental.pallas import tpu_sc as plsc

N = 10000
E = 320000
E2 = E // 2
NC = 2
NS = 16
NW = NC * NS
C = 128
NP = 10240
RT = NP // NS

_PREC = jax.lax.Precision.DEFAULT


def _dot(a, b):
    return jnp.dot(a, b, precision=_PREC, preferred_element_type=jnp.float32)


def _elu(x):
    return jnp.where(x > 0, x, jnp.exp(jnp.minimum(x, 0.0)) - 1.0)


def _ln_rows(x, g, b, eps=1e-5):
    m = jnp.mean(x, axis=-1, keepdims=True)
    v = jnp.mean((x - m) ** 2, axis=-1, keepdims=True)
    return (x - m) / jnp.sqrt(v + eps) * g + b


def _node1_body(x_ref, w1ab, wsk, wg, bsk, bg, g0, b0,
                xln_ref, t1tab_ref, skip_ref, gate_ref):
    x = _ln_rows(x_ref[...], g0[...], b0[...])
    xln_ref[...] = x
    t1tab_ref[...] = _dot(x, w1ab[...])
    skip = _dot(x, wsk[...]) + bsk[...]
    skip_ref[...] = skip
    gate_ref[...] = jax.nn.sigmoid(_dot(skip, wg[...]) + bg[...])


def _edge_body(ea_ref, we1, we2, we3, wc1, wc2, w1c, w2c,
               be1, be2, be3, bc1, bc2, bm11, bm21, gee, bee,
               e_ref, t1_ref, t2_ref):
    ea = ea_ref[...]
    el = _ln_rows(ea, gee[...], bee[...])
    e1 = jax.nn.relu(_dot(el, we1[...]) + be1[...])
    e2 = jax.nn.relu(_dot(e1, we2[...]) + be2[...])
    e3 = _dot(e2, we3[...]) + be3[...]
    w = jax.nn.relu(_dot(ea, wc1[...]) + bc1[...])
    w = jax.nn.sigmoid(_dot(w, wc2[...]) + bc2[...])
    e = e3 * w
    e_ref[...] = e
    t1_ref[...] = _dot(e, w1c[...]) + bm11[...]
    t2_ref[...] = _dot(e, w2c[...]) + bm21[...]


def _c1_body(gd_ref, gs_ref, t1_ref, wm12, bm12, h2x_ref):
    h1 = jax.nn.relu(gd_ref[:, 0:64] + gs_ref[:, 64:128] + t1_ref[...])
    h2x_ref[:, 0:64] = jax.nn.relu(_dot(h1, wm12[...]) + bm12[...])
    be = h1.shape[0]
    col = lax.broadcasted_iota(jnp.int32, (be, 64), 1)
    h2x_ref[:, 64:128] = jnp.where(col == 0, 1.0, 0.0)


def _node2_body(s1a_ref, s1b_ref, wm13, bm13, g1, b1, w2a, w2b,
                qd_ref, qs_ref):
    s1 = (s1a_ref[0, :, 0:64] + s1a_ref[1, :, 0:64]
          + s1b_ref[0, :, 0:64] + s1b_ref[1, :, 0:64])
    cnt = (s1a_ref[0, :, 64:65] + s1a_ref[1, :, 64:65]
           + s1b_ref[0, :, 64:65] + s1b_ref[1, :, 64:65])
    deg = jnp.maximum(cnt, 1.0)
    x1p = (_dot(s1, wm13[...]) + cnt * bm13[...]) / deg
    x1 = jax.nn.leaky_relu(_ln_rows(x1p, g1[...], b1[...]), 0.01)
    qd_ref[...] = _dot(x1, w2a[...])
    qs_ref[...] = _dot(x1, w2b[...])


def _c2_body(gd_ref, gs_ref, t2_ref, wm22, bm22, g2_ref):
    g1 = jax.nn.relu(gd_ref[...] + gs_ref[...] + t2_ref[...])
    g2_ref[...] = jax.nn.relu(_dot(g1, wm22[...]) + bm22[...])


def _final_body(s1a_ref, s1b_ref, s2a_ref, s2b_ref, s3a_ref, s3b_ref,
                skip_ref, gate_ref,
                wm23, bm23, g2, b2, wp1a, wp1b, bp1, wp2, bp2, wp3, bp3,
                xfc_ref, pr_ref):
    cnt = (s1a_ref[0, :, 64:65] + s1a_ref[1, :, 64:65]
           + s1b_ref[0, :, 64:65] + s1b_ref[1, :, 64:65])
    deg = jnp.maximum(cnt, 1.0)
    s2 = s2a_ref[0] + s2a_ref[1] + s2b_ref[0] + s2b_ref[1]
    x2p = (_dot(s2, wm23[...]) + cnt * bm23[...]) / deg
    x2 = jax.nn.relu(_ln_rows(x2p, g2[...], b2[...]))
    skip = skip_ref[...]
    gate = gate_ref[...]
    xf = gate * skip + (1.0 - gate) * x2
    efm = (s3a_ref[0] + s3a_ref[1] + s3b_ref[0] + s3b_ref[1]) / deg
    xfc_ref[:, 0:128] = xf
    xfc_ref[:, 128:256] = efm
    h = _elu(_dot(xf, wp1a[...]) + _dot(efm, wp1b[...]) + bp1[...])
    h = _elu(_dot(h, wp2[...]) + bp2[...])
    pr_ref[...] = _dot(h, wp3[...]) + bp3[...]


def _full(shape):
    nd = len(shape)
    return pl.BlockSpec(shape, lambda i, _nd=nd: (0,) * _nd)


def _rows(bs, width, off=0):
    return pl.BlockSpec((bs, width), lambda i, _o=off: (i + _o, 0))


def _part(bn):
    return pl.BlockSpec((NC, bn, 128), lambda i: (0, i, 0))


def _worker_ids():
    c = lax.axis_index("c")
    s = lax.axis_index("s")
    return c, s


def _per_worker(ne):
    return ((ne + NW * C - 1) // (NW * C)) * C


def _n_chunks(base_local, ne, ew):
    left = jnp.maximum(jnp.minimum(ew, ne - base_local), 0)
    return left // C


def _gather_body(e0, ne, td, ts, idxd, idxs, od, os_,
                 idxvd, idxvs, rowsd, rowss, semd, sems):
    c, s = _worker_ids()
    ew = _per_worker(ne)
    base = (s * NC + c) * ew

    def chunk(j, carry):
        off = base + j * C
        goff = e0 + off
        pltpu.sync_copy(idxd.at[pl.ds(goff, C)], idxvd)
        gd = pltpu.async_copy(td.at[idxvd], rowsd, semd)
        pltpu.sync_copy(idxs.at[pl.ds(goff, C)], idxvs)
        gs = pltpu.async_copy(ts.at[idxvs], rowss, sems)
        gd.wait()
        pltpu.sync_copy(rowsd, od.at[pl.ds(off, C)])
        gs.wait()
        pltpu.sync_copy(rowss, os_.at[pl.ds(off, C)])
        return carry

    lax.fori_loop(0, _n_chunks(base, ne, ew), chunk, 0)


def _make_gather(e0, ne):
    mesh = plsc.VectorSubcoreMesh(core_axis_name="c", subcore_axis_name="s")
    return pl.kernel(
        functools.partial(_gather_body, e0, ne),
        out_type=(jax.ShapeDtypeStruct((ne, 128), jnp.float32),
                  jax.ShapeDtypeStruct((ne, 128), jnp.float32)),
        mesh=mesh,
        scratch_types=[pltpu.VMEM((C,), jnp.int32),
                       pltpu.VMEM((C,), jnp.int32),
                       pltpu.VMEM((C, 128), jnp.float32),
                       pltpu.VMEM((C, 128), jnp.float32),
                       pltpu.SemaphoreType.DMA, pltpu.SemaphoreType.DMA],
    )


def _scatter_body(e0, ne, h, idx, out, acc, buf, idxv, semh):
    c, s = _worker_ids()
    ew = _per_worker(ne)
    base = (s * NC + c) * ew

    z = jnp.zeros((16,), jnp.float32)

    def zb(i, carry):
        for k in range(128 // 16):
            buf[i, pl.ds(k * 16, 16)] = z
        return carry

    lax.fori_loop(0, C, zb, 0)
    row0 = s * RT
    done = 0
    while done < RT:
        step = min(C, RT - done)
        pltpu.sync_copy(buf.at[pl.ds(0, step)], acc.at[pl.ds(row0 + done, step)])
        done += step
    plsc.subcore_barrier()

    def chunk(j, carry):
        off = base + j * C
        lh = pltpu.async_copy(h.at[pl.ds(off, C)], buf, semh)
        pltpu.sync_copy(idx.at[pl.ds(e0 + off, C)], idxv)
        lh.wait()
        pltpu.sync_copy(buf, acc.at[idxv], add=True)
        return carry

    lax.fori_loop(0, _n_chunks(base, ne, ew), chunk, 0)
    plsc.subcore_barrier()

    out_row = c * NP + s * RT
    done = 0
    while done < RT:
        step = min(C, RT - done)
        pltpu.sync_copy(acc.at[pl.ds(row0 + done, step)],
                        out.at[pl.ds(out_row + done, step)])
        done += step


def _make_scatter(e0, ne):
    mesh = plsc.VectorSubcoreMesh(core_axis_name="c", subcore_axis_name="s")
    return pl.kernel(
        functools.partial(_scatter_body, e0, ne),
        out_type=jax.ShapeDtypeStruct((NC * NP, 128), jnp.float32),
        mesh=mesh,
        scratch_types=[pltpu.VMEM_SHARED((NP, 128), jnp.float32),
                       pltpu.VMEM((C, 128), jnp.float32),
                       pltpu.VMEM((C,), jnp.int32),
                       pltpu.SemaphoreType.DMA],
    )


_gather_a = _make_gather(0, E2)
_gather_b = _make_gather(E2, E2)
_scatter_a = _make_scatter(0, E2)
_scatter_b = _make_scatter(E2, E2)


def kernel(x_in, edge_index, edge_attr, params):
    p = params
    x = x_in[0]
    ea = edge_attr[0]
    src = edge_index[0, 0].astype(jnp.int32)
    dst = edge_index[0, 1].astype(jnp.int32)

    w1ab = jnp.concatenate([p['W_m11'][:, 0:128].T,
                            p['W_m11'][:, 128:256].T], axis=1)
    w1c = p['W_m11'][:, 256:384].T
    w2a = p['W_m21'][:, 0:64].T
    w2b = p['W_m21'][:, 64:128].T
    w2c = p['W_m21'][:, 128:256].T

    def r1(v):
        return v.reshape(1, -1)

    BN, BE = 2000, 4000
    gn = N // BN
    ge = E // BE
    ge2 = E2 // BE
    ob = E2 // BE

    xln, t1tab, skip, gate = pl.pallas_call(
        _node1_body,
        grid=(gn,),
        in_specs=[_rows(BN, 128), _full((128, 128)),
                  _full((128, 128)), _full((128, 128)), _full((1, 128)),
                  _full((1, 128)), _full((1, 128)), _full((1, 128))],
        out_specs=[_rows(BN, 128), _rows(BN, 128),
                   _rows(BN, 128), _rows(BN, 128)],
        out_shape=[jax.ShapeDtypeStruct((N, 128), jnp.float32),
                   jax.ShapeDtypeStruct((N, 128), jnp.float32),
                   jax.ShapeDtypeStruct((N, 128), jnp.float32),
                   jax.ShapeDtypeStruct((N, 128), jnp.float32)],
    )(x, w1ab, p['W_skip'].T, p['W_g'].T, r1(p['b_skip']), r1(p['b_g']),
      r1(p['g0']), r1(p['b0']))

    def edge_chunk(ea_h):
        return pl.pallas_call(
            _edge_body,
            grid=(ge2,),
            in_specs=[_rows(BE, 16), _full((16, 128)), _full((128, 256)),
                      _full((256, 128)), _full((16, 16)), _full((16, 1)),
                      _full((128, 64)), _full((128, 128)),
                      _full((1, 128)), _full((1, 256)), _full((1, 128)),
                      _full((1, 16)), _full((1, 1)), _full((1, 64)),
                      _full((1, 128)), _full((1, 16)), _full((1, 16))],
            out_specs=[_rows(BE, 128), _rows(BE, 64), _rows(BE, 128)],
            out_shape=[jax.ShapeDtypeStruct((E2, 128), jnp.float32),
                       jax.ShapeDtypeStruct((E2, 64), jnp.float32),
                       jax.ShapeDtypeStruct((E2, 128), jnp.float32)],
        )(ea_h, p['W_e1'].T, p['W_e2'].T, p['W_e3'].T, p['W_c1'].T,
          p['W_c2'].T, w1c, w2c, r1(p['b_e1']), r1(p['b_e2']), r1(p['b_e3']),
          r1(p['b_c1']), r1(p['b_c2']), r1(p['b_m11']), r1(p['b_m21']),
          r1(p['g_ee']), r1(p['b_ee']))

    ea_half = (ea[:E2], ea[E2:])

    def c1_chunk(gd, gs, t1h):
        return pl.pallas_call(
            _c1_body,
            grid=(ge2,),
            in_specs=[_rows(BE, 128), _rows(BE, 128), _rows(BE, 64),
                      _full((64, 64)), _full((1, 64))],
            out_specs=_rows(BE, 128),
            out_shape=jax.ShapeDtypeStruct((E2, 128), jnp.float32),
        )(gd, gs, t1h, p['W_m12'].T, r1(p['b_m12']))

    gd1a, gs1a = _gather_a(t1tab, t1tab, dst, src)
    ea_res = [edge_chunk(h) for h in ea_half]
    gd1b, gs1b = _gather_b(t1tab, t1tab, dst, src)
    h2xa = c1_chunk(gd1a, gs1a, ea_res[0][1])
    h2xb = c1_chunk(gd1b, gs1b, ea_res[1][1])
    s1pa = _scatter_a(h2xa, dst).reshape(NC, NP, 128)
    s1pb = _scatter_b(h2xb, dst).reshape(NC, NP, 128)

    qd, qs = pl.pallas_call(
        _node2_body,
        grid=(gn,),
        in_specs=[_part(BN), _part(BN),
                  _full((64, 64)), _full((1, 64)), _full((1, 64)),
                  _full((1, 64)), _full((64, 128)), _full((64, 128))],
        out_specs=[_rows(BN, 128), _rows(BN, 128)],
        out_shape=[jax.ShapeDtypeStruct((N, 128), jnp.float32),
                   jax.ShapeDtypeStruct((N, 128), jnp.float32)],
    )(s1pa, s1pb, p['W_m13'].T, r1(p['b_m13']), r1(p['g1']), r1(p['b1']),
      w2a, w2b)

    def c2_chunk(gd, gs, t2h):
        return pl.pallas_call(
            _c2_body,
            grid=(ge2,),
            in_specs=[_rows(BE, 128), _rows(BE, 128), _rows(BE, 128),
                      _full((128, 128)), _full((1, 128))],
            out_specs=_rows(BE, 128),
            out_shape=jax.ShapeDtypeStruct((E2, 128), jnp.float32),
        )(gd, gs, t2h, p['W_m22'].T, r1(p['b_m22']))

    gd2a, gs2a = _gather_a(qd, qs, dst, src)
    gd2b, gs2b = _gather_b(qd, qs, dst, src)
    g2a = c2_chunk(gd2a, gs2a, ea_res[0][2])
    g2b = c2_chunk(gd2b, gs2b, ea_res[1][2])
    s2pa = _scatter_a(g2a, dst).reshape(NC, NP, 128)
    s2pb = _scatter_b(g2b, dst).reshape(NC, NP, 128)
    s3pa = _scatter_a(ea_res[0][0], dst).reshape(NC, NP, 128)
    s3pb = _scatter_b(ea_res[1][0], dst).reshape(NC, NP, 128)

    xfc, pr = pl.pallas_call(
        _final_body,
        grid=(gn,),
        in_specs=[_part(BN), _part(BN), _part(BN), _part(BN), _part(BN),
                  _part(BN),
                  _rows(BN, 128), _rows(BN, 128),
                  _full((128, 128)), _full((1, 128)), _full((1, 128)),
                  _full((1, 128)), _full((128, 128)), _full((128, 128)),
                  _full((1, 128)), _full((128, 64)), _full((1, 64)),
                  _full((64, 1)), _full((1, 1))],
        out_specs=[_rows(BN, 256), _rows(BN, 1)],
        out_shape=[jax.ShapeDtypeStruct((N, 256), jnp.float32),
                   jax.ShapeDtypeStruct((N, 1), jnp.float32)],
    )(s1pa, s1pb, s2pa, s2pb, s3pa, s3pb, skip, gate, p['W_m23'].T,
      r1(p['b_m23']),
      r1(p['g2']), r1(p['b2']), p['W_p1'][:, 0:128].T,
      p['W_p1'][:, 128:256].T, r1(p['b_p1']), p['W_p2'].T, r1(p['b_p2']),
      p['W_p3'].T, r1(p['b_p3']))

    return (xfc[None], pr[None])


# speedup vs baseline: 1.0779x; 1.0779x over previous
import functools

import jax
import jax.numpy as jnp
from jax import lax
from jax.experimental import pallas as pl
from jax.experimental.pallas import tpu as pltpu
from jax.experimental.pallas import tpu_sc as plsc

N = 10000
E = 320000
E2 = E // 2
NC = 2
NS = 16
NW = NC * NS
C = 128
NP = 10240
RT = NP // NS

_PREC = jax.lax.Precision.DEFAULT


def _dot(a, b):
    return jnp.dot(a, b, precision=_PREC, preferred_element_type=jnp.float32)


def _elu(x):
    return jnp.where(x > 0, x, jnp.exp(jnp.minimum(x, 0.0)) - 1.0)


def _ln_rows(x, g, b, eps=1e-5):
    m = jnp.mean(x, axis=-1, keepdims=True)
    v = jnp.mean((x - m) ** 2, axis=-1, keepdims=True)
    return (x - m) / jnp.sqrt(v + eps) * g + b


def _node1_body(x_ref, w1ab, wsk, wg, bsk, bg, g0, b0,
                xln_ref, t1tab_ref, skip_ref, gate_ref):
    x = _ln_rows(x_ref[...], g0[...], b0[...])
    xln_ref[...] = x
    t1tab_ref[...] = _dot(x, w1ab[...])
    skip = _dot(x, wsk[...]) + bsk[...]
    skip_ref[...] = skip
    gate_ref[...] = jax.nn.sigmoid(_dot(skip, wg[...]) + bg[...])


def _edge_body(ea_ref, we1, we2, we3, wc1, wc2, w1c, w2c,
               be1, be2, be3, bc1, bc2, bm11, bm21, gee, bee,
               e_ref, t1_ref, t2_ref):
    ea = ea_ref[...]
    el = _ln_rows(ea, gee[...], bee[...])
    e1 = jax.nn.relu(_dot(el, we1[...]) + be1[...])
    e2 = jax.nn.relu(_dot(e1, we2[...]) + be2[...])
    e3 = _dot(e2, we3[...]) + be3[...]
    w = jax.nn.relu(_dot(ea, wc1[...]) + bc1[...])
    w = jax.nn.sigmoid(_dot(w, wc2[...]) + bc2[...])
    e = e3 * w
    e_ref[...] = e
    t1_ref[...] = _dot(e, w1c[...]) + bm11[...]
    t2_ref[...] = _dot(e, w2c[...]) + bm21[...]


def _c1_body(gd_ref, gs_ref, t1_ref, wm12, bm12, h2x_ref):
    h1 = jax.nn.relu(gd_ref[:, 0:64] + gs_ref[:, 64:128] + t1_ref[...])
    h2x_ref[:, 0:64] = jax.nn.relu(_dot(h1, wm12[...]) + bm12[...])
    be = h1.shape[0]
    col = lax.broadcasted_iota(jnp.int32, (be, 64), 1)
    h2x_ref[:, 64:128] = jnp.where(col == 0, 1.0, 0.0)


def _node2_body(s1a_ref, s1b_ref, wm13, bm13, g1, b1, w2a, w2b,
                qd_ref, qs_ref):
    s1 = (s1a_ref[0, :, 0:64] + s1a_ref[1, :, 0:64]
          + s1b_ref[0, :, 0:64] + s1b_ref[1, :, 0:64])
    cnt = (s1a_ref[0, :, 64:65] + s1a_ref[1, :, 64:65]
           + s1b_ref[0, :, 64:65] + s1b_ref[1, :, 64:65])
    deg = jnp.maximum(cnt, 1.0)
    x1p = (_dot(s1, wm13[...]) + cnt * bm13[...]) / deg
    x1 = jax.nn.leaky_relu(_ln_rows(x1p, g1[...], b1[...]), 0.01)
    qd_ref[...] = _dot(x1, w2a[...])
    qs_ref[...] = _dot(x1, w2b[...])


def _c2_body(gd_ref, gs_ref, t2_ref, wm22, bm22, g2_ref):
    g1 = jax.nn.relu(gd_ref[...] + gs_ref[...] + t2_ref[...])
    g2_ref[...] = jax.nn.relu(_dot(g1, wm22[...]) + bm22[...])


def _final_body(s1a_ref, s1b_ref, s2a_ref, s2b_ref, s3_ref,
                skip_ref, gate_ref,
                wm23, bm23, g2, b2, wp1a, wp1b, bp1, wp2, bp2, wp3, bp3,
                xfc_ref, pr_ref):
    cnt = (s1a_ref[0, :, 64:65] + s1a_ref[1, :, 64:65]
           + s1b_ref[0, :, 64:65] + s1b_ref[1, :, 64:65])
    deg = jnp.maximum(cnt, 1.0)
    s2 = s2a_ref[0] + s2a_ref[1] + s2b_ref[0] + s2b_ref[1]
    x2p = (_dot(s2, wm23[...]) + cnt * bm23[...]) / deg
    x2 = jax.nn.relu(_ln_rows(x2p, g2[...], b2[...]))
    skip = skip_ref[...]
    gate = gate_ref[...]
    xf = gate * skip + (1.0 - gate) * x2
    efm = (s3_ref[0] + s3_ref[1]) / deg
    xfc_ref[:, 0:128] = xf
    xfc_ref[:, 128:256] = efm
    h = _elu(_dot(xf, wp1a[...]) + _dot(efm, wp1b[...]) + bp1[...])
    h = _elu(_dot(h, wp2[...]) + bp2[...])
    pr_ref[...] = _dot(h, wp3[...]) + bp3[...]


def _full(shape):
    nd = len(shape)
    return pl.BlockSpec(shape, lambda i, _nd=nd: (0,) * _nd)


def _rows(bs, width, off=0):
    return pl.BlockSpec((bs, width), lambda i, _o=off: (i + _o, 0))


def _part(bn):
    return pl.BlockSpec((NC, bn, 128), lambda i: (0, i, 0))


def _worker_ids():
    c = lax.axis_index("c")
    s = lax.axis_index("s")
    return c, s


def _per_worker(ne):
    return ((ne + NW * C - 1) // (NW * C)) * C


def _n_chunks(base_local, ne, ew):
    left = jnp.maximum(jnp.minimum(ew, ne - base_local), 0)
    return left // C


def _gather_body(e0, ne, td, ts, idxd, idxs, od, os_,
                 idxvd, idxvs, rowsd, rowss, semd, sems):
    c, s = _worker_ids()
    ew = _per_worker(ne)
    base = (s * NC + c) * ew

    def chunk(j, carry):
        off = base + j * C
        goff = e0 + off
        pltpu.sync_copy(idxd.at[pl.ds(goff, C)], idxvd)
        gd = pltpu.async_copy(td.at[idxvd], rowsd, semd)
        pltpu.sync_copy(idxs.at[pl.ds(goff, C)], idxvs)
        gs = pltpu.async_copy(ts.at[idxvs], rowss, sems)
        gd.wait()
        pltpu.sync_copy(rowsd, od.at[pl.ds(off, C)])
        gs.wait()
        pltpu.sync_copy(rowss, os_.at[pl.ds(off, C)])
        return carry

    lax.fori_loop(0, _n_chunks(base, ne, ew), chunk, 0)


def _make_gather(e0, ne):
    mesh = plsc.VectorSubcoreMesh(core_axis_name="c", subcore_axis_name="s")
    return pl.kernel(
        functools.partial(_gather_body, e0, ne),
        out_type=(jax.ShapeDtypeStruct((ne, 128), jnp.float32),
                  jax.ShapeDtypeStruct((ne, 128), jnp.float32)),
        mesh=mesh,
        scratch_types=[pltpu.VMEM((C,), jnp.int32),
                       pltpu.VMEM((C,), jnp.int32),
                       pltpu.VMEM((C, 128), jnp.float32),
                       pltpu.VMEM((C, 128), jnp.float32),
                       pltpu.SemaphoreType.DMA, pltpu.SemaphoreType.DMA],
    )


def _scatter_body(e0, ne, h, idx, out, acc, buf, idxv, semh):
    c, s = _worker_ids()
    ew = _per_worker(ne)
    base = (s * NC + c) * ew

    z = jnp.zeros((16,), jnp.float32)

    def zb(i, carry):
        for k in range(128 // 16):
            buf[i, pl.ds(k * 16, 16)] = z
        return carry

    lax.fori_loop(0, C, zb, 0)
    row0 = s * RT
    done = 0
    while done < RT:
        step = min(C, RT - done)
        pltpu.sync_copy(buf.at[pl.ds(0, step)], acc.at[pl.ds(row0 + done, step)])
        done += step
    plsc.subcore_barrier()

    def chunk(j, carry):
        off = base + j * C
        lh = pltpu.async_copy(h.at[pl.ds(off, C)], buf, semh)
        pltpu.sync_copy(idx.at[pl.ds(e0 + off, C)], idxv)
        lh.wait()
        pltpu.sync_copy(buf, acc.at[idxv], add=True)
        return carry

    lax.fori_loop(0, _n_chunks(base, ne, ew), chunk, 0)
    plsc.subcore_barrier()

    out_row = c * NP + s * RT
    done = 0
    while done < RT:
        step = min(C, RT - done)
        pltpu.sync_copy(acc.at[pl.ds(row0 + done, step)],
                        out.at[pl.ds(out_row + done, step)])
        done += step


def _make_scatter(e0, ne):
    mesh = plsc.VectorSubcoreMesh(core_axis_name="c", subcore_axis_name="s")
    return pl.kernel(
        functools.partial(_scatter_body, e0, ne),
        out_type=jax.ShapeDtypeStruct((NC * NP, 128), jnp.float32),
        mesh=mesh,
        scratch_types=[pltpu.VMEM_SHARED((NP, 128), jnp.float32),
                       pltpu.VMEM((C, 128), jnp.float32),
                       pltpu.VMEM((C,), jnp.int32),
                       pltpu.SemaphoreType.DMA],
    )


_gather_a = _make_gather(0, E2)
_gather_b = _make_gather(E2, E2)
_scatter_a = _make_scatter(0, E2)
_scatter_b = _make_scatter(E2, E2)
_scatter_full = _make_scatter(0, E)


def kernel(x_in, edge_index, edge_attr, params):
    p = params
    x = x_in[0]
    ea = edge_attr[0]
    src = edge_index[0, 0].astype(jnp.int32)
    dst = edge_index[0, 1].astype(jnp.int32)

    w1ab = jnp.concatenate([p['W_m11'][:, 0:128].T,
                            p['W_m11'][:, 128:256].T], axis=1)
    w1c = p['W_m11'][:, 256:384].T
    w2a = p['W_m21'][:, 0:64].T
    w2b = p['W_m21'][:, 64:128].T
    w2c = p['W_m21'][:, 128:256].T

    def r1(v):
        return v.reshape(1, -1)

    BN, BE = 2000, 4000
    gn = N // BN
    ge = E // BE
    ge2 = E2 // BE
    ob = E2 // BE

    xln, t1tab, skip, gate = pl.pallas_call(
        _node1_body,
        grid=(gn,),
        in_specs=[_rows(BN, 128), _full((128, 128)),
                  _full((128, 128)), _full((128, 128)), _full((1, 128)),
                  _full((1, 128)), _full((1, 128)), _full((1, 128))],
        out_specs=[_rows(BN, 128), _rows(BN, 128),
                   _rows(BN, 128), _rows(BN, 128)],
        out_shape=[jax.ShapeDtypeStruct((N, 128), jnp.float32),
                   jax.ShapeDtypeStruct((N, 128), jnp.float32),
                   jax.ShapeDtypeStruct((N, 128), jnp.float32),
                   jax.ShapeDtypeStruct((N, 128), jnp.float32)],
    )(x, w1ab, p['W_skip'].T, p['W_g'].T, r1(p['b_skip']), r1(p['b_g']),
      r1(p['g0']), r1(p['b0']))

    e, t1, t2 = pl.pallas_call(
        _edge_body,
        grid=(ge,),
        in_specs=[_rows(BE, 16), _full((16, 128)), _full((128, 256)),
                  _full((256, 128)), _full((16, 16)), _full((16, 1)),
                  _full((128, 64)), _full((128, 128)),
                  _full((1, 128)), _full((1, 256)), _full((1, 128)),
                  _full((1, 16)), _full((1, 1)), _full((1, 64)),
                  _full((1, 128)), _full((1, 16)), _full((1, 16))],
        out_specs=[_rows(BE, 128), _rows(BE, 64), _rows(BE, 128)],
        out_shape=[jax.ShapeDtypeStruct((E, 128), jnp.float32),
                   jax.ShapeDtypeStruct((E, 64), jnp.float32),
                   jax.ShapeDtypeStruct((E, 128), jnp.float32)],
    )(ea, p['W_e1'].T, p['W_e2'].T, p['W_e3'].T, p['W_c1'].T, p['W_c2'].T,
      w1c, w2c, r1(p['b_e1']), r1(p['b_e2']), r1(p['b_e3']), r1(p['b_c1']),
      r1(p['b_c2']), r1(p['b_m11']), r1(p['b_m21']), r1(p['g_ee']),
      r1(p['b_ee']))

    def c1_chunk(gd, gs, off):
        return pl.pallas_call(
            _c1_body,
            grid=(ge2,),
            in_specs=[_rows(BE, 128), _rows(BE, 128), _rows(BE, 64, off),
                      _full((64, 64)), _full((1, 64))],
            out_specs=_rows(BE, 128),
            out_shape=jax.ShapeDtypeStruct((E2, 128), jnp.float32),
        )(gd, gs, t1, p['W_m12'].T, r1(p['b_m12']))

    gd1a, gs1a = _gather_a(t1tab, t1tab, dst, src)
    gd1b, gs1b = _gather_b(t1tab, t1tab, dst, src)
    h2xa = c1_chunk(gd1a, gs1a, 0)
    h2xb = c1_chunk(gd1b, gs1b, ob)
    s1pa = _scatter_a(h2xa, dst).reshape(NC, NP, 128)
    s1pb = _scatter_b(h2xb, dst).reshape(NC, NP, 128)

    qd, qs = pl.pallas_call(
        _node2_body,
        grid=(gn,),
        in_specs=[_part(BN), _part(BN),
                  _full((64, 64)), _full((1, 64)), _full((1, 64)),
                  _full((1, 64)), _full((64, 128)), _full((64, 128))],
        out_specs=[_rows(BN, 128), _rows(BN, 128)],
        out_shape=[jax.ShapeDtypeStruct((N, 128), jnp.float32),
                   jax.ShapeDtypeStruct((N, 128), jnp.float32)],
    )(s1pa, s1pb, p['W_m13'].T, r1(p['b_m13']), r1(p['g1']), r1(p['b1']),
      w2a, w2b)

    def c2_chunk(gd, gs, off):
        return pl.pallas_call(
            _c2_body,
            grid=(ge2,),
            in_specs=[_rows(BE, 128), _rows(BE, 128), _rows(BE, 128, off),
                      _full((128, 128)), _full((1, 128))],
            out_specs=_rows(BE, 128),
            out_shape=jax.ShapeDtypeStruct((E2, 128), jnp.float32),
        )(gd, gs, t2, p['W_m22'].T, r1(p['b_m22']))

    gd2a, gs2a = _gather_a(qd, qs, dst, src)
    gd2b, gs2b = _gather_b(qd, qs, dst, src)
    g2a = c2_chunk(gd2a, gs2a, 0)
    g2b = c2_chunk(gd2b, gs2b, ob)
    s2pa = _scatter_a(g2a, dst).reshape(NC, NP, 128)
    s2pb = _scatter_b(g2b, dst).reshape(NC, NP, 128)
    s3p = _scatter_full(e, dst).reshape(NC, NP, 128)

    xfc, pr = pl.pallas_call(
        _final_body,
        grid=(gn,),
        in_specs=[_part(BN), _part(BN), _part(BN), _part(BN), _part(BN),
                  _rows(BN, 128), _rows(BN, 128),
                  _full((128, 128)), _full((1, 128)), _full((1, 128)),
                  _full((1, 128)), _full((128, 128)), _full((128, 128)),
                  _full((1, 128)), _full((128, 64)), _full((1, 64)),
                  _full((64, 1)), _full((1, 1))],
        out_specs=[_rows(BN, 256), _rows(BN, 1)],
        out_shape=[jax.ShapeDtypeStruct((N, 256), jnp.float32),
                   jax.ShapeDtypeStruct((N, 1), jnp.float32)],
    )(s1pa, s1pb, s2pa, s2pb, s3p, skip, gate, p['W_m23'].T, r1(p['b_m23']),
      r1(p['g2']), r1(p['b2']), p['W_p1'][:, 0:128].T,
      p['W_p1'][:, 128:256].T, r1(p['b_p1']), p['W_p2'].T, r1(p['b_p2']),
      p['W_p3'].T, r1(p['b_p3']))

    return (xfc[None], pr[None])
